# baseline (speedup 1.0000x reference)
import functools

import jax
import jax.numpy as jnp
import numpy as np
from jax import lax
from jax.experimental import pallas as pl
from jax.experimental.pallas import tpu as pltpu, tpu_sc as plsc

B, D, N = 8, 256, 128
R = B * D
NC, NS, L = 2, 16, 16
NW = NC * NS
RPW = R // NW
NCH = N // L

_mesh = plsc.VectorSubcoreMesh(core_axis_name="c", subcore_axis_name="s")
_splat_dnums = lax.GatherDimensionNumbers(
    offset_dims=(), collapsed_slice_dims=(0,), start_index_map=(0,))


@functools.partial(
    pl.kernel,
    mesh=_mesh,
    out_type=jax.ShapeDtypeStruct((R, N, N), jnp.float32),
    scratch_types=[
        pltpu.VMEM((RPW, N), jnp.float32),
        pltpu.VMEM((2, N, N), jnp.float32),
        pltpu.SemaphoreType.DMA,
        pltpu.SemaphoreType.DMA,
        pltpu.SemaphoreType.DMA,
    ],
)
def _band_max(x_hbm, out_hbm, x_v, buf, sem0, sem1, semx):
    wid = lax.axis_index("s") * NC + lax.axis_index("c")
    base = wid * RPW
    xcopy = pltpu.make_async_copy(x_hbm.at[pl.ds(base, RPW)], x_v, semx)
    xcopy.start()

    lane = lax.iota(jnp.int32, L)
    zero = jnp.zeros((L,), jnp.float32)
    ninf = jnp.full((L,), -jnp.inf, jnp.float32)

    def zero_fill(b2):
        for g in range(1, NCH):
            def zbody(r, c, g=g, b2=b2):
                for ch in range(g):
                    buf[b2, r, ch * L:(ch + 1) * L] = zero
                return c
            lax.fori_loop(g * L, (g + 1) * L, zbody, 0)

    zero_fill(0)
    xcopy.wait()

    def compute_tile(i, b2):
        carry = [ninf] * NCH
        for g in range(NCH - 1, -1, -1):
            xg = x_v[i, g * L:(g + 1) * L]
            for j in range(L - 1, -1, -1):
                s = g * L + j
                vv = lax.gather(
                    xg, jnp.full((L, 1), j, jnp.int32), _splat_dnums,
                    slice_sizes=(1,),
                    mode=lax.GatherScatterMode.PROMISE_IN_BOUNDS)
                m = lane >= j
                if j == L - 1:
                    carry[g] = jnp.where(m, vv, ninf)
                elif j == 0:
                    carry[g] = jnp.maximum(carry[g], vv)
                else:
                    carry[g] = jnp.maximum(carry[g], jnp.where(m, vv, ninf))
                sval = carry[g] if j == 0 else jnp.where(m, carry[g], zero)
                buf[b2, s, g * L:(g + 1) * L] = sval
                for c in range(g + 1, NCH):
                    carry[c] = jnp.maximum(carry[c], vv)
                    buf[b2, s, c * L:(c + 1) * L] = carry[c]

    def gbody(i, c):
        b2 = jnp.bitwise_and(i, 1)

        @pl.when(jnp.logical_and(i > 1, b2 == 0))
        def _():
            pltpu.make_async_copy(buf.at[0], out_hbm.at[0], sem0).wait()

        @pl.when(jnp.logical_and(i > 1, b2 == 1))
        def _():
            pltpu.make_async_copy(buf.at[1], out_hbm.at[0], sem1).wait()

        @pl.when(i == 1)
        def _():
            zero_fill(1)

        compute_tile(i, b2)

        @pl.when(b2 == 0)
        def _():
            pltpu.make_async_copy(buf.at[0], out_hbm.at[base + i], sem0).start()

        @pl.when(b2 == 1)
        def _():
            pltpu.make_async_copy(buf.at[1], out_hbm.at[base + i], sem1).start()
        return c
    lax.fori_loop(0, RPW, gbody, 0)
    pltpu.make_async_copy(buf.at[0], out_hbm.at[0], sem0).wait()
    pltpu.make_async_copy(buf.at[1], out_hbm.at[0], sem1).wait()


_MASK2D = np.triu(np.ones((N, N), dtype=bool))


def kernel(x):
    map2d = _band_max(x.reshape(R, N))
    return map2d.reshape(B, D, N, N), jnp.asarray(_MASK2D)


# speedup vs baseline: 1.0244x; 1.0244x over previous
import functools

import jax
import jax.numpy as jnp
import numpy as np
from jax import lax
from jax.experimental import pallas as pl
from jax.experimental.pallas import tpu as pltpu, tpu_sc as plsc

B, D, N = 8, 256, 128
R = B * D
NC, NS, L = 2, 16, 16
NW = NC * NS
RPW = R // NW
NCH = N // L

_mesh = plsc.VectorSubcoreMesh(core_axis_name="c", subcore_axis_name="s")
_splat_dnums = lax.GatherDimensionNumbers(
    offset_dims=(), collapsed_slice_dims=(0,), start_index_map=(0,))


@functools.partial(
    pl.kernel,
    mesh=_mesh,
    out_type=jax.ShapeDtypeStruct((R, N, N), jnp.float32),
    scratch_types=[
        pltpu.VMEM((RPW, N), jnp.float32),
        pltpu.VMEM((2, N, N), jnp.float32),
        pltpu.SemaphoreType.DMA,
        pltpu.SemaphoreType.DMA,
        pltpu.SemaphoreType.DMA,
    ],
)
def _band_max(x_hbm, out_hbm, x_v, buf, sem0, sem1, semx):
    wid = lax.axis_index("s") * NC + lax.axis_index("c")
    base = wid * RPW
    xcopy = pltpu.make_async_copy(x_hbm.at[pl.ds(base, RPW)], x_v, semx)
    xcopy.start()

    lane = lax.iota(jnp.int32, L)
    zero = jnp.zeros((L,), jnp.float32)
    ninf = jnp.full((L,), -jnp.inf, jnp.float32)

    def zbody(r, c):
        for b2 in range(2):
            for ch in range(NCH):
                buf[b2, r, ch * L:(ch + 1) * L] = zero
        return c
    lax.fori_loop(0, N, zbody, 0)
    xcopy.wait()

    def compute_tile(i, b2):
        carry = [ninf] * NCH
        for g in range(NCH - 1, -1, -1):
            xg = x_v[i, g * L:(g + 1) * L]
            for j in range(L - 1, -1, -1):
                s = g * L + j
                vv = lax.gather(
                    xg, jnp.full((L, 1), j, jnp.int32), _splat_dnums,
                    slice_sizes=(1,),
                    mode=lax.GatherScatterMode.PROMISE_IN_BOUNDS)
                m = lane >= j
                if j == L - 1:
                    carry[g] = jnp.where(m, vv, ninf)
                elif j == 0:
                    carry[g] = jnp.maximum(carry[g], vv)
                else:
                    carry[g] = jnp.maximum(carry[g], jnp.where(m, vv, ninf))
                sval = carry[g] if j == 0 else jnp.where(m, carry[g], zero)
                buf[b2, s, g * L:(g + 1) * L] = sval
                for c in range(g + 1, NCH):
                    carry[c] = jnp.maximum(carry[c], vv)
                    buf[b2, s, c * L:(c + 1) * L] = carry[c]

    def gbody(i, c):
        b2 = jnp.bitwise_and(i, 1)

        @pl.when(jnp.logical_and(i > 1, b2 == 0))
        def _():
            pltpu.make_async_copy(buf.at[0], out_hbm.at[0], sem0).wait()

        @pl.when(jnp.logical_and(i > 1, b2 == 1))
        def _():
            pltpu.make_async_copy(buf.at[1], out_hbm.at[0], sem1).wait()

        compute_tile(i, b2)

        @pl.when(b2 == 0)
        def _():
            pltpu.make_async_copy(buf.at[0], out_hbm.at[base + i], sem0).start()

        @pl.when(b2 == 1)
        def _():
            pltpu.make_async_copy(buf.at[1], out_hbm.at[base + i], sem1).start()
        return c
    lax.fori_loop(0, RPW, gbody, 0)
    pltpu.make_async_copy(buf.at[0], out_hbm.at[0], sem0).wait()
    pltpu.make_async_copy(buf.at[1], out_hbm.at[0], sem1).wait()


_MASK2D = np.triu(np.ones((N, N), dtype=bool))


def kernel(x):
    map2d = _band_max(x.reshape(R, N))
    return map2d.reshape(B, D, N, N), jnp.asarray(_MASK2D)
